# baseline (speedup 1.0000x reference)
import jax
import jax.numpy as jnp
from jax.experimental import pallas as pl

_TB = 16
_SB = 64
_C = 256
_TP = 64
_SP = 1024
_W = 32
_SIGMA2 = 4.0


def _xcorr_kernel(t_ref, s_ref, o_ref):
    t = t_ref[0]
    s = s_ref[0]
    xc = jnp.dot(t, s, preferred_element_type=jnp.float32)
    mag_t = jnp.sqrt(jnp.sum(t * t, axis=1, keepdims=True))
    mag_s = jnp.sqrt(jnp.sum(s * s, axis=0, keepdims=True))
    xc = xc / (mag_t * mag_s + 1e-8)

    m = jnp.max(xc, axis=1, keepdims=True)
    j = jax.lax.broadcasted_iota(jnp.int32, (_TP, _SP), 1)
    idx = jnp.min(jnp.where(xc == m, j, _SP), axis=1, keepdims=True)

    fidx = idx.astype(jnp.float32)
    py = fidx * (1.0 / _W)
    px = (idx & (_W - 1)).astype(jnp.float32)

    y = (j >> 5).astype(jnp.float32)
    x = (j & (_W - 1)).astype(jnp.float32)
    dy = y - py
    dx = x - px
    g = jnp.exp((-0.5 / _SIGMA2) * (dy * dy + dx * dx))
    o_ref[0] = xc * g


def kernel(template, search):
    tmat = template.reshape(_TB, _C, _TP).transpose(0, 2, 1)
    smat = search.reshape(_SB, _C, _SP)

    out = pl.pallas_call(
        _xcorr_kernel,
        grid=(_SB,),
        in_specs=[
            pl.BlockSpec((1, _TP, _C), lambda b: (b % _TB, 0, 0)),
            pl.BlockSpec((1, _C, _SP), lambda b: (b, 0, 0)),
        ],
        out_specs=pl.BlockSpec((1, _TP, _SP), lambda b: (b, 0, 0)),
        out_shape=jax.ShapeDtypeStruct((_SB, _TP, _SP), jnp.float32),
    )(tmat, smat)

    return out.reshape(_SB, _TP, _W, _W)


# speedup vs baseline: 1.2977x; 1.2977x over previous
import jax
import jax.numpy as jnp
from jax.experimental import pallas as pl

_TB = 16
_SB = 64
_C = 256
_TP = 64
_SP = 1024
_W = 32
_SIGMA2 = 4.0
_BLK = 8


def _xcorr_kernel(t_ref, s_ref, o_ref):
    t = t_ref[...]
    s = s_ref[...]
    xc = jax.lax.dot_general(
        t, s, (((2,), (1,)), ((0,), (0,))),
        preferred_element_type=jnp.float32)
    mag_t = jnp.sqrt(jnp.sum(t * t, axis=2, keepdims=True))
    mag_s = jnp.sqrt(jnp.sum(s * s, axis=1, keepdims=True))
    xc = xc / (mag_t * mag_s + 1e-8)

    m = jnp.max(xc, axis=2, keepdims=True)
    j = jax.lax.broadcasted_iota(jnp.int32, (_BLK, _TP, _SP), 2)
    idx = jnp.min(jnp.where(xc == m, j, _SP), axis=2, keepdims=True)

    fidx = idx.astype(jnp.float32)
    py = fidx * (1.0 / _W)
    px = (idx & (_W - 1)).astype(jnp.float32)

    y = (j >> 5).astype(jnp.float32)
    x = (j & (_W - 1)).astype(jnp.float32)
    dy = y - py
    dx = x - px
    g = jnp.exp((-0.5 / _SIGMA2) * (dy * dy + dx * dx))
    o_ref[...] = xc * g


def kernel(template, search):
    tmat = template.reshape(_TB, _C, _TP).transpose(0, 2, 1)
    smat = search.reshape(_SB, _C, _SP)

    out = pl.pallas_call(
        _xcorr_kernel,
        grid=(_SB // _BLK,),
        in_specs=[
            pl.BlockSpec((_BLK, _TP, _C), lambda b: (b % (_TB // _BLK), 0, 0)),
            pl.BlockSpec((_BLK, _C, _SP), lambda b: (b, 0, 0)),
        ],
        out_specs=pl.BlockSpec((_BLK, _TP, _SP), lambda b: (b, 0, 0)),
        out_shape=jax.ShapeDtypeStruct((_SB, _TP, _SP), jnp.float32),
    )(tmat, smat)

    return out.reshape(_SB, _TP, _W, _W)


# speedup vs baseline: 2.6988x; 2.0797x over previous
import jax
import jax.numpy as jnp
from jax.experimental import pallas as pl

_TB = 16
_SB = 64
_C = 256
_TP = 64
_SP = 1024
_W = 32
_SIGMA2 = 4.0
_BLK = 8


def _xcorr_kernel(t_ref, s_ref, o_ref):
    t = t_ref[...]
    s = s_ref[...]
    xc = jax.lax.dot_general(
        s, t, (((2,), (2,)), ((0,), (0,))),
        preferred_element_type=jnp.float32)
    ones = jnp.ones((_BLK, 1, _C), jnp.float32)
    mt2 = jax.lax.dot_general(
        ones, t * t, (((2,), (2,)), ((0,), (0,))),
        preferred_element_type=jnp.float32)
    mag_t = jnp.sqrt(mt2)
    mag_s = jnp.sqrt(jnp.sum(s * s, axis=2, keepdims=True))
    xc = xc / (mag_t * mag_s + 1e-8)

    m = jnp.max(xc, axis=1, keepdims=True)
    j = jax.lax.broadcasted_iota(jnp.int32, (_BLK, _SP, _TP), 1)
    idx = jnp.min(jnp.where(xc == m, j, _SP), axis=1, keepdims=True)

    fidx = idx.astype(jnp.float32)
    py = fidx * (1.0 / _W)
    px = (idx & (_W - 1)).astype(jnp.float32)

    y = (j >> 5).astype(jnp.float32)
    x = (j & (_W - 1)).astype(jnp.float32)
    dy = y - py
    dx = x - px
    g = jnp.exp((-0.5 / _SIGMA2) * (dy * dy + dx * dx))
    o_ref[...] = xc * g


def kernel(template, search):
    t = template.transpose(0, 2, 3, 1).reshape(_TB, _TP, _C)
    s = search.transpose(0, 2, 3, 1).reshape(_SB, _SP, _C)

    out = pl.pallas_call(
        _xcorr_kernel,
        grid=(_SB // _BLK,),
        in_specs=[
            pl.BlockSpec((_BLK, _TP, _C), lambda b: (b % (_TB // _BLK), 0, 0)),
            pl.BlockSpec((_BLK, _SP, _C), lambda b: (b, 0, 0)),
        ],
        out_specs=pl.BlockSpec((_BLK, _SP, _TP), lambda b: (b, 0, 0)),
        out_shape=jax.ShapeDtypeStruct((_SB, _SP, _TP), jnp.float32),
    )(t, s)

    return out.reshape(_SB, _W, _W, _TP).transpose(0, 3, 1, 2)


# speedup vs baseline: 3.5531x; 1.3165x over previous
import jax
import jax.numpy as jnp
from jax.experimental import pallas as pl

_TB = 16
_SB = 64
_C = 256
_TP = 64
_SP = 1024
_W = 32
_SIGMA2 = 4.0
_BLK = 8
_PK = _BLK // 2


def _xcorr_kernel(t_ref, s_ref, o_ref):
    t = t_ref[...]
    s = s_ref[...]
    s_e = s[:, 0]
    s_o = s[:, 1]
    dn = (((2,), (2,)), ((0,), (0,)))
    m_e = jax.lax.dot_general(s_e, t, dn, preferred_element_type=jnp.float32)
    m_o = jax.lax.dot_general(s_o, t, dn, preferred_element_type=jnp.float32)
    lane = jax.lax.broadcasted_iota(jnp.int32, (1, 1, 2 * _TP), 2)
    emask = lane < _TP
    xc = jnp.where(emask, m_e, m_o)

    mt2 = jax.lax.dot_general(
        jnp.ones((_PK, 1, _C), jnp.float32), t * t, dn,
        preferred_element_type=jnp.float32)
    s2 = (s * s).reshape(2 * _PK, _SP, _C)
    ms2 = jax.lax.dot_general(
        s2, jnp.ones((2 * _PK, _C, 1), jnp.float32),
        (((2,), (1,)), ((0,), (0,))),
        preferred_element_type=jnp.float32).reshape(_PK, 2, _SP, 1)
    ms2p = jnp.where(emask, ms2[:, 0], ms2[:, 1])
    norm = jnp.sqrt(mt2) * jnp.sqrt(ms2p) + 1e-8
    xc = xc / norm

    m = jnp.max(xc, axis=1, keepdims=True)
    j = jax.lax.broadcasted_iota(jnp.int32, (_PK, _SP, 2 * _TP), 1)
    idx = jnp.min(jnp.where(xc == m, j, _SP), axis=1, keepdims=True)

    fidx = idx.astype(jnp.float32)
    py = fidx * (1.0 / _W)
    px = (idx & (_W - 1)).astype(jnp.float32)

    y = (j >> 5).astype(jnp.float32)
    x = (j & (_W - 1)).astype(jnp.float32)
    dy = y - py
    dx = x - px
    g = jnp.exp((-0.5 / _SIGMA2) * (dy * dy + dx * dx))
    res = xc * g
    for p in range(_PK):
        o_ref[2 * p] = res[p, :, 0:_TP]
        o_ref[2 * p + 1] = res[p, :, _TP:2 * _TP]


def kernel(template, search):
    t = template.transpose(0, 2, 3, 1).reshape(_TB // 2, 2 * _TP, _C)
    s = search.transpose(0, 2, 3, 1).reshape(_SB // 2, 2, _SP, _C)

    out = pl.pallas_call(
        _xcorr_kernel,
        grid=(_SB // _BLK,),
        in_specs=[
            pl.BlockSpec((_PK, 2 * _TP, _C), lambda b: (b % 2, 0, 0)),
            pl.BlockSpec((_PK, 2, _SP, _C), lambda b: (b, 0, 0, 0)),
        ],
        out_specs=pl.BlockSpec((_BLK, _SP, _TP), lambda b: (b, 0, 0)),
        out_shape=jax.ShapeDtypeStruct((_SB, _SP, _TP), jnp.float32),
    )(t, s)

    return out.reshape(_SB, _W, _W, _TP).transpose(0, 3, 1, 2)


# speedup vs baseline: 3.7760x; 1.0627x over previous
import jax
import jax.numpy as jnp
from jax.experimental import pallas as pl

_TB = 16
_SB = 64
_C = 256
_TP = 64
_SP = 1024
_W = 32
_SIGMA2 = 4.0
_BLK = 8
_PK = _BLK // 2


def _xcorr_kernel(t_ref, s_ref, o_ref):
    t = t_ref[...]
    s = s_ref[...]
    s_e = s[:, 0]
    s_o = s[:, 1]
    dn = (((2,), (2,)), ((0,), (0,)))
    m_e = jax.lax.dot_general(s_e, t, dn, preferred_element_type=jnp.float32)
    m_o = jax.lax.dot_general(s_o, t, dn, preferred_element_type=jnp.float32)
    lane = jax.lax.broadcasted_iota(jnp.int32, (1, 1, 2 * _TP), 2)
    emask = lane < _TP
    xc = jnp.where(emask, m_e, m_o)

    mt2 = jax.lax.dot_general(
        jnp.ones((_PK, 1, _C), jnp.float32), t * t, dn,
        preferred_element_type=jnp.float32)
    s2 = (s * s).reshape(2 * _PK, _SP, _C)
    ms2 = jax.lax.dot_general(
        s2, jnp.ones((2 * _PK, _C, 1), jnp.float32),
        (((2,), (1,)), ((0,), (0,))),
        preferred_element_type=jnp.float32).reshape(_PK, 2, _SP, 1)
    ms2p = jnp.where(emask, ms2[:, 0], ms2[:, 1])
    xc = xc * jax.lax.rsqrt(mt2 * ms2p)

    j = jax.lax.broadcasted_iota(jnp.int32, (1, _SP, 2 * _TP), 1)
    idx = jnp.argmax(xc, axis=1)[:, None, :]

    fidx = idx.astype(jnp.float32)
    py = fidx * (1.0 / _W)
    px = (idx & (_W - 1)).astype(jnp.float32)

    y = (j >> 5).astype(jnp.float32)
    x = (j & (_W - 1)).astype(jnp.float32)
    a = (-0.5 / _SIGMA2) * (y * y + x * x)
    b = (-0.5 / _SIGMA2) * (py * py + px * px)
    c = (1.0 / _SIGMA2) * py
    d = (1.0 / _SIGMA2) * px
    g = jnp.exp(a + b + y * c + x * d)
    res = xc * g
    for p in range(_PK):
        o_ref[2 * p] = res[p, :, 0:_TP]
        o_ref[2 * p + 1] = res[p, :, _TP:2 * _TP]


def kernel(template, search):
    t = template.transpose(0, 2, 3, 1).reshape(_TB // 2, 2 * _TP, _C)
    s = search.transpose(0, 2, 3, 1).reshape(_SB // 2, 2, _SP, _C)

    out = pl.pallas_call(
        _xcorr_kernel,
        grid=(_SB // _BLK,),
        in_specs=[
            pl.BlockSpec((_PK, 2 * _TP, _C), lambda b: (b % 2, 0, 0)),
            pl.BlockSpec((_PK, 2, _SP, _C), lambda b: (b, 0, 0, 0)),
        ],
        out_specs=pl.BlockSpec((_BLK, _SP, _TP), lambda b: (b, 0, 0)),
        out_shape=jax.ShapeDtypeStruct((_SB, _SP, _TP), jnp.float32),
    )(t, s)

    return out.reshape(_SB, _W, _W, _TP).transpose(0, 3, 1, 2)


# speedup vs baseline: 3.8870x; 1.0294x over previous
import jax
import jax.numpy as jnp
from jax.experimental import pallas as pl

_TB = 16
_SB = 64
_C = 256
_TP = 64
_SP = 1024
_W = 32
_SIGMA2 = 4.0
_BLK = 8
_PK = _BLK // 2


def _xcorr_kernel(t_ref, s_ref, o_ref):
    t = t_ref[...]
    s = s_ref[...]
    s_e = s[:, 0]
    s_o = s[:, 1]
    dn = (((2,), (2,)), ((0,), (0,)))
    m_e = jax.lax.dot_general(s_e, t, dn, preferred_element_type=jnp.float32)
    m_o = jax.lax.dot_general(s_o, t, dn, preferred_element_type=jnp.float32)
    lane = jax.lax.broadcasted_iota(jnp.int32, (1, 1, 2 * _TP), 2)
    emask = lane < _TP
    xc = jnp.where(emask, m_e, m_o)

    mt2 = jnp.sum(t * t, axis=2, keepdims=True)
    mt2 = jnp.transpose(mt2, (0, 2, 1))
    ms2 = jnp.sum(s * s, axis=3, keepdims=True)
    ms2p = jnp.where(emask, ms2[:, 0], ms2[:, 1])
    norm = jnp.sqrt(mt2) * jnp.sqrt(ms2p) + 1e-8
    xc = xc / norm

    m = jnp.max(xc, axis=1, keepdims=True)
    j = jax.lax.broadcasted_iota(jnp.int32, (_PK, _SP, 2 * _TP), 1)
    idx = jnp.min(jnp.where(xc == m, j, _SP), axis=1, keepdims=True)

    fidx = idx.astype(jnp.float32)
    py = fidx * (1.0 / _W)
    px = (idx & (_W - 1)).astype(jnp.float32)

    y = (j >> 5).astype(jnp.float32)
    x = (j & (_W - 1)).astype(jnp.float32)
    dy = y - py
    dx = x - px
    g = jnp.exp((-0.5 / _SIGMA2) * (dy * dy + dx * dx))
    res = xc * g
    for p in range(_PK):
        o_ref[2 * p] = res[p, :, 0:_TP]
        o_ref[2 * p + 1] = res[p, :, _TP:2 * _TP]


def kernel(template, search):
    t = template.transpose(0, 2, 3, 1).reshape(_TB // 2, 2 * _TP, _C)
    s = search.transpose(0, 2, 3, 1).reshape(_SB // 2, 2, _SP, _C)

    out = pl.pallas_call(
        _xcorr_kernel,
        grid=(_SB // _BLK,),
        in_specs=[
            pl.BlockSpec((_PK, 2 * _TP, _C), lambda b: (b % 2, 0, 0)),
            pl.BlockSpec((_PK, 2, _SP, _C), lambda b: (b, 0, 0, 0)),
        ],
        out_specs=pl.BlockSpec((_BLK, _SP, _TP), lambda b: (b, 0, 0)),
        out_shape=jax.ShapeDtypeStruct((_SB, _SP, _TP), jnp.float32),
    )(t, s)

    return out.reshape(_SB, _W, _W, _TP).transpose(0, 3, 1, 2)
